# baseline (speedup 1.0000x reference)
import functools

import jax
import jax.numpy as jnp
from jax import lax
from jax.experimental import pallas as pl
from jax.experimental.pallas import tpu as pltpu
from jax.experimental.pallas import tpu_sc as plsc

N = 10000
NP = 10240
E = 160000
E2 = 170000
EPa = 172032
EC = 512
TN = 128
NT = NP // TN
B = 512
SIDE_N = 994
SIDE_NP = 1024
SIDE_D = 1024
Z1P = 1664
NW = 32
RPW = EPa // NW

LAYERS = ((128, 768, 8, 96), (768, 1024, 8, 128), (1024, 1024, 8, 128),
          (1024, 128, 1, 128))

_f32 = jnp.float32
_i32 = jnp.int32
_bf16 = jnp.bfloat16


def _att_mats(a_s, a_d, F, heads, ocw):
    rows = jnp.arange(F, dtype=_i32) // ocw
    As = jnp.zeros((F, 16), _f32).at[jnp.arange(F), rows].set(a_s.reshape(F))
    Ad = jnp.zeros((F, 16), _f32).at[jnp.arange(F), rows].set(a_d.reshape(F))
    return As, Ad


def _packed(F):
    return F >= 256


def _gw(F):
    return F // 2 if _packed(F) else F


def _gdt(F):
    return _i32 if _packed(F) else _f32


def _pack(h, F):
    if not _packed(F):
        return h
    au = lax.bitcast_convert_type(
        h[:, :F // 2].astype(jnp.bfloat16).astype(_f32), jnp.uint32)
    bu = lax.bitcast_convert_type(
        h[:, F // 2:].astype(jnp.bfloat16).astype(_f32), jnp.uint32)
    return lax.bitcast_convert_type((au >> 16) | bu, _i32)


def _unpack(vv, F):
    vu = lax.bitcast_convert_type(vv, jnp.uint32)
    af = lax.bitcast_convert_type(vu << 16, _f32)
    bf = lax.bitcast_convert_type(vu & jnp.uint32(0xFFFF0000), _f32)
    return jnp.concatenate([af, bf], axis=1)


def _rep_mat(F, ocw):
    return (jnp.arange(F, dtype=_i32)[None, :] // ocw
            == jnp.arange(16, dtype=_i32)[:, None]).astype(_f32)


def _make_mm_first(K, F, interpret=False):
    def body(x_ref, w_ref, ad_ref, hb_ref, ald_ref):
        h = jnp.dot(x_ref[...].astype(_bf16), w_ref[...].astype(_bf16),
                    preferred_element_type=_f32)
        hb_ref[...] = _pack(h, F)
        ald_ref[...] = jnp.dot(h, ad_ref[...], preferred_element_type=_f32)

    return pl.pallas_call(
        body,
        grid=(NP // 256,),
        in_specs=[
            pl.BlockSpec((256, K), lambda i: (i, 0)),
            pl.BlockSpec((K, F), lambda i: (0, 0)),
            pl.BlockSpec((F, 16), lambda i: (0, 0)),
        ],
        out_specs=[
            pl.BlockSpec((256, _gw(F)), lambda i: (i, 0)),
            pl.BlockSpec((256, 16), lambda i: (i, 0)),
        ],
        out_shape=[
            jax.ShapeDtypeStruct((NP, _gw(F)), _gdt(F)),
            jax.ShapeDtypeStruct((NP, 16), _f32),
        ],
        interpret=interpret,
    )


def _make_mm_next(K, F, interpret=False):
    def body(u_ref, s_ref, r_ref, b_ref, w_ref, ad_ref, hb_ref, ald_ref):
        sexp = jnp.dot(s_ref[...], r_ref[...],
                       preferred_element_type=_f32) + 1e-16
        xv = jnp.maximum(u_ref[...] / sexp + b_ref[...], 0.0)
        h = jnp.dot(xv.astype(_bf16), w_ref[...].astype(_bf16),
                    preferred_element_type=_f32)
        hb_ref[...] = _pack(h, F)
        ald_ref[...] = jnp.dot(h, ad_ref[...], preferred_element_type=_f32)

    return pl.pallas_call(
        body,
        grid=(NP // 256,),
        in_specs=[
            pl.BlockSpec((256, K), lambda i: (i, 0)),
            pl.BlockSpec((256, 16), lambda i: (i, 0)),
            pl.BlockSpec((16, K), lambda i: (0, 0)),
            pl.BlockSpec((1, K), lambda i: (0, 0)),
            pl.BlockSpec((K, F), lambda i: (0, 0)),
            pl.BlockSpec((F, 16), lambda i: (0, 0)),
        ],
        out_specs=[
            pl.BlockSpec((256, _gw(F)), lambda i: (i, 0)),
            pl.BlockSpec((256, 16), lambda i: (i, 0)),
        ],
        out_shape=[
            jax.ShapeDtypeStruct((NP, _gw(F)), _gdt(F)),
            jax.ShapeDtypeStruct((NP, 16), _f32),
        ],
        interpret=interpret,
    )


def _sc_mesh():
    return plsc.VectorSubcoreMesh(core_axis_name="c", subcore_axis_name="s",
                                  num_cores=2, num_subcores=16)


def _sc_gather_rows(table, srcs, F, G, dtype=_f32):
    nch = RPW // G
    assert nch % 2 == 0
    npairs = nch // 2

    def body(tab_hbm, src_hbm, out_hbm, idx0, idx1, rows0, rows1, sem0, sem1):
        wid = lax.axis_index("s") * 2 + lax.axis_index("c")
        base = wid * RPW

        pltpu.sync_copy(src_hbm.at[pl.ds(base, G)], idx0)
        pltpu.async_copy(tab_hbm.at[idx0], rows0, sem0)

        def pair(cc, carry):
            c = cc * 2
            b0 = base + c * G
            b1 = b0 + G
            pltpu.make_async_copy(tab_hbm.at[idx0], rows0, sem0).wait()
            pltpu.sync_copy(src_hbm.at[pl.ds(b1, G)], idx1)
            pltpu.async_copy(tab_hbm.at[idx1], rows1, sem1)
            pltpu.sync_copy(rows0, out_hbm.at[pl.ds(b0, G)])
            pltpu.make_async_copy(tab_hbm.at[idx1], rows1, sem1).wait()

            @pl.when(cc + 1 < npairs)
            def _():
                b2 = b0 + 2 * G
                pltpu.sync_copy(src_hbm.at[pl.ds(b2, G)], idx0)
                pltpu.async_copy(tab_hbm.at[idx0], rows0, sem0)

            pltpu.sync_copy(rows1, out_hbm.at[pl.ds(b1, G)])
            return carry

        lax.fori_loop(0, npairs, pair, 0)

    f = pl.kernel(
        body,
        out_type=jax.ShapeDtypeStruct((EPa, F), dtype),
        mesh=_sc_mesh(),
        scratch_types=[
            pltpu.VMEM((G,), _i32),
            pltpu.VMEM((G,), _i32),
            pltpu.VMEM((G, F), dtype),
            pltpu.VMEM((G, F), dtype),
            pltpu.SemaphoreType.DMA,
            pltpu.SemaphoreType.DMA,
        ],
    )
    return f(table, srcs)


def _make_seg(F, interpret=False):
    def body(off_ref, dst_hbm, v_hbm, ald_ref, as_ref, r_ref, u_ref, s_ref,
             dstv, vv, uacc, sacc, sem1, sem2):
        t = pl.program_id(0)
        start = (off_ref[t] // EC) * EC
        end = off_ref[t + 1]
        nch = (end - start + EC - 1) // EC
        rowids = t * TN + lax.broadcasted_iota(_i32, (TN, EC), 0)
        uacc[...] = jnp.zeros((TN, F), _f32)
        sacc[...] = jnp.zeros((TN, 16), _f32)
        rmat = r_ref[...].astype(_bf16)
        aldt = ald_ref[...].astype(_bf16)
        asm = as_ref[...].astype(_bf16)

        def chunk(c, carry):
            base = pl.multiple_of(start + c * EC, EC)
            cd = pltpu.make_async_copy(dst_hbm.at[:, pl.ds(base, EC)], dstv,
                                       sem1)
            cv = pltpu.make_async_copy(v_hbm.at[pl.ds(base, EC), :], vv, sem2)
            cd.start()
            cv.start()
            cd.wait()
            cv.wait()
            vf = _unpack(vv[...], F) if _packed(F) else vv[...]
            vb = vf.astype(_bf16)
            maskb = (rowids == dstv[...]).astype(_bf16)
            als_e = jnp.dot(vb, asm, preferred_element_type=_f32)
            ald_e = lax.dot_general(maskb, aldt, (((0,), (0,)), ((), ())),
                                    preferred_element_type=_f32)
            alpha = als_e + ald_e
            alpha = jnp.where(alpha >= 0, alpha, 0.2 * alpha)
            e = jnp.exp(alpha)
            eb = e.astype(_bf16)
            sacc[...] += jnp.dot(maskb, eb, preferred_element_type=_f32)
            eexp = jnp.dot(eb, rmat, preferred_element_type=_f32)
            vw = (vf * eexp).astype(_bf16)
            uacc[...] += jnp.dot(maskb, vw, preferred_element_type=_f32)
            return carry

        lax.fori_loop(0, nch, chunk, 0)
        u_ref[...] = uacc[...]
        s_ref[...] = sacc[...]

    grid_spec = pltpu.PrefetchScalarGridSpec(
        num_scalar_prefetch=1,
        grid=(NT,),
        in_specs=[
            pl.BlockSpec(memory_space=pltpu.MemorySpace.HBM),
            pl.BlockSpec(memory_space=pltpu.MemorySpace.HBM),
            pl.BlockSpec((TN, 16), lambda i, *_: (i, 0)),
            pl.BlockSpec((F, 16), lambda i, *_: (0, 0)),
            pl.BlockSpec((16, F), lambda i, *_: (0, 0)),
        ],
        out_specs=[
            pl.BlockSpec((TN, F), lambda i, *_: (i, 0)),
            pl.BlockSpec((TN, 16), lambda i, *_: (i, 0)),
        ],
        scratch_shapes=[
            pltpu.VMEM((1, EC), _i32),
            pltpu.VMEM((EC, _gw(F)), _gdt(F)),
            pltpu.VMEM((TN, F), _f32),
            pltpu.VMEM((TN, 16), _f32),
            pltpu.SemaphoreType.DMA,
            pltpu.SemaphoreType.DMA,
        ],
    )
    return pl.pallas_call(
        body,
        grid_spec=grid_spec,
        out_shape=[
            jax.ShapeDtypeStruct((NP, F), _f32),
            jax.ShapeDtypeStruct((NP, 16), _f32),
        ],
        interpret=interpret,
    )


def _make_pool(interpret=False):
    def body(boff_ref, u_ref, s_ref, b_ref, g_ref, xs):
        sden = s_ref[:, 0:1] + 1e-16
        xs[...] = jnp.maximum(u_ref[...] / sden + b_ref[...], 0.0)
        iota8 = lax.broadcasted_iota(_i32, (8, 128), 0)

        def g_body(g, carry):
            st = boff_ref[g]
            en = boff_ref[g + 1]
            nch = (en - st + 7) // 8

            def c_body(c, acc):
                r0 = st + c * 8
                rows = xs[pl.ds(r0, 8), :]
                valid = (r0 + iota8) < en
                return jnp.maximum(acc, jnp.where(valid, rows, -1e30))

            acc = lax.fori_loop(0, nch, c_body,
                                jnp.full((8, 128), -1e30, _f32))
            g_ref[pl.ds(g, 1), :] = jnp.max(acc, axis=0, keepdims=True)
            return carry

        lax.fori_loop(0, B, g_body, 0)

    grid_spec = pltpu.PrefetchScalarGridSpec(
        num_scalar_prefetch=1,
        grid=(1,),
        in_specs=[
            pl.BlockSpec((NP, 128), lambda i, *_: (0, 0)),
            pl.BlockSpec((NP, 16), lambda i, *_: (0, 0)),
            pl.BlockSpec((1, 128), lambda i, *_: (0, 0)),
        ],
        out_specs=[pl.BlockSpec((B, 128), lambda i, *_: (0, 0))],
        scratch_shapes=[pltpu.VMEM((NP, 128), _f32)],
    )
    return pl.pallas_call(
        body,
        grid_spec=grid_spec,
        out_shape=[jax.ShapeDtypeStruct((B, 128), _f32)],
        interpret=interpret,
    )


def _make_mlp(interpret=False):
    def body(g_ref, w_ref, z_ref, se_ref, W5, b5, W6, b6, Ww1, bw1, Ww2, bw2,
             Wz1, bz1, Wz2, bz2, Ws1, bs1, Ws2, bs2, freq_ref, dv_ref,
             sv_ref):
        dot = lambda a, bb: jnp.dot(a, bb, preferred_element_type=_f32)
        relu = lambda v: jnp.maximum(v, 0.0)
        g2 = relu(dot(relu(dot(g_ref[...], W5[...]) + b5[...]), W6[...])
                  + b6[...])
        wv = relu(dot(relu(dot(w_ref[...], Ww1[...]) + bw1[...]), Ww2[...])
                  + bw2[...])
        zv = relu(dot(relu(dot(z_ref[...], Wz1[...]) + bz1[...]), Wz2[...])
                  + bz2[...])
        xwz = jnp.tanh(jnp.maximum(jnp.maximum(g2, wv), zv))
        dv = xwz / jnp.maximum(
            jnp.sqrt(jnp.sum(xwz * xwz, axis=1, keepdims=True)), 1e-12)
        sd = jnp.tanh(dot(relu(dot(se_ref[...], Ws1[...]) + bs1[...]),
                          Ws2[...]) + bs2[...])
        sv = sd / jnp.maximum(
            jnp.sqrt(jnp.sum(sd * sd, axis=1, keepdims=True)), 1e-12)
        freq_ref[...] = 5.0 * lax.dot_general(
            dv, sv, (((1,), (1,)), ((), ())), preferred_element_type=_f32)
        dv_ref[...] = dv
        sv_ref[...] = sv

    full = lambda shape: pl.BlockSpec(shape, lambda: tuple(0 for _ in shape))
    in_shapes = [(B, 128), (B, 256), (B, 512), (SIDE_NP, SIDE_D),
                 (128, 64), (1, 64), (64, 64), (1, 64),
                 (256, 256), (1, 256), (256, 64), (1, 64),
                 (512, Z1P), (1, Z1P), (Z1P, 64), (1, 64),
                 (SIDE_D, 64), (1, 64), (64, 64), (1, 64)]
    return pl.pallas_call(
        body,
        in_specs=[full(s) for s in in_shapes],
        out_specs=[full((B, SIDE_NP)), full((B, 64)), full((SIDE_NP, 64))],
        out_shape=[
            jax.ShapeDtypeStruct((B, SIDE_NP), _f32),
            jax.ShapeDtypeStruct((B, 64), _f32),
            jax.ShapeDtypeStruct((SIDE_NP, 64), _f32),
        ],
        interpret=interpret,
    )


def kernel(x, edge_index, batch, w, z, side_effects, params):
    p = params
    loop = jnp.arange(N, dtype=_i32)
    src = jnp.concatenate([edge_index[0].astype(_i32), loop])
    dst = jnp.concatenate([edge_index[1].astype(_i32), loop])
    order = jnp.argsort(dst)
    srcs = src[order]
    dsts = dst[order]
    pad = jnp.full((EPa - E2,), NP - 1, _i32)
    srcs_p = jnp.concatenate([srcs, pad])
    dsts_p = jnp.concatenate([dsts, pad])
    off = jnp.searchsorted(dsts, jnp.arange(NT + 1) * TN).astype(_i32)
    dst2d = dsts_p.reshape(1, EPa)
    boff = jnp.searchsorted(batch, jnp.arange(B + 1)).astype(_i32)
    xp = jnp.zeros((NP, 128), _f32).at[:N].set(x)

    Wl = (p['W1'], p['W2'], p['W3'], p['W4'])
    Al = ((p['a1s'], p['a1d']), (p['a2s'], p['a2d']), (p['a3s'], p['a3d']),
          (p['a4s'], p['a4d']))
    bl = (p['b1'], p['b2'], p['b3'], p['b4'])

    u = s = None
    for li, (K, F, heads, ocw) in enumerate(LAYERS):
        As, Ad = _att_mats(Al[li][0], Al[li][1], F, heads, ocw)
        R = _rep_mat(F, ocw)
        if li == 0:
            hb, ald = _make_mm_first(K, F)(xp, Wl[0], Ad)
        else:
            Rprev = _rep_mat(K, LAYERS[li - 1][3])
            hb, ald = _make_mm_next(K, F)(
                u, s, Rprev, bl[li - 1].reshape(1, K), Wl[li], Ad)
        G = 112 if F >= 1024 else (168 if F >= 512 else 448)
        V = _sc_gather_rows(hb, srcs_p, _gw(F), G, _gdt(F))
        u, s = _make_seg(F)(off, dst2d, V, ald, As, R)

    g = _make_pool()(boff, u, s, bl[3].reshape(1, 128))[0]

    Wz1p = jnp.zeros((512, Z1P), _f32).at[:, :1600].set(p['Wz1'])
    bz1p = jnp.zeros((1, Z1P), _f32).at[:, :1600].set(p['bz1'][None])
    Wz2p = jnp.zeros((Z1P, 64), _f32).at[:1600].set(p['Wz2'])
    sep = jnp.zeros((SIDE_NP, SIDE_D), _f32).at[:SIDE_N].set(side_effects)

    freq_p, dv, sv_p = _make_mlp()(
        g, w, z, sep,
        p['W5'], p['b5'].reshape(1, 64), p['W6'], p['b6'].reshape(1, 64),
        p['Ww1'], p['bw1'].reshape(1, 256), p['Ww2'], p['bw2'].reshape(1, 64),
        Wz1p, bz1p, Wz2p, p['bz2'].reshape(1, 64),
        p['Ws1'], p['bs1'].reshape(1, 64), p['Ws2'], p['bs2'].reshape(1, 64))
    return freq_p[:, :SIDE_N], dv, sv_p[:SIDE_N]


# speedup vs baseline: 1.3295x; 1.3295x over previous
import functools

import jax
import jax.numpy as jnp
from jax import lax
from jax.experimental import pallas as pl
from jax.experimental.pallas import tpu as pltpu
from jax.experimental.pallas import tpu_sc as plsc

N = 10000
NP = 10240
E = 160000
E2 = 170000
EPa = 172032
EC = 512
TN = 128
NT = NP // TN
B = 512
SIDE_N = 994
SIDE_NP = 1024
SIDE_D = 1024
Z1P = 1664
NW = 32
RPW = EPa // NW

LAYERS = ((128, 768, 8, 96), (768, 1024, 8, 128), (1024, 1024, 8, 128),
          (1024, 128, 1, 128))

_f32 = jnp.float32
_i32 = jnp.int32
_bf16 = jnp.bfloat16


def _att_mats(a_s, a_d, F, heads, ocw):
    rows = jnp.arange(F, dtype=_i32) // ocw
    As = jnp.zeros((F, 16), _f32).at[jnp.arange(F), rows].set(a_s.reshape(F))
    Ad = jnp.zeros((F, 16), _f32).at[jnp.arange(F), rows].set(a_d.reshape(F))
    return As, Ad


def _packed(F):
    return F >= 256


def _gw(F):
    return F // 2 if _packed(F) else F


def _gdt(F):
    return _i32 if _packed(F) else _f32


def _pack(h, F):
    if not _packed(F):
        return h
    au = lax.bitcast_convert_type(
        h[:, :F // 2].astype(jnp.bfloat16).astype(_f32), jnp.uint32)
    bu = lax.bitcast_convert_type(
        h[:, F // 2:].astype(jnp.bfloat16).astype(_f32), jnp.uint32)
    return lax.bitcast_convert_type((au >> 16) | bu, _i32)


def _unpack(vv, F):
    vu = lax.bitcast_convert_type(vv, jnp.uint32)
    af = lax.bitcast_convert_type(vu << 16, _f32)
    bf = lax.bitcast_convert_type(vu & jnp.uint32(0xFFFF0000), _f32)
    return jnp.concatenate([af, bf], axis=1)


def _rep_mat(F, ocw):
    return (jnp.arange(F, dtype=_i32)[None, :] // ocw
            == jnp.arange(16, dtype=_i32)[:, None]).astype(_f32)


def _make_mm_first(K, F, interpret=False):
    def body(x_ref, w_ref, ad_ref, hb_ref, ald_ref):
        h = jnp.dot(x_ref[...].astype(_bf16), w_ref[...].astype(_bf16),
                    preferred_element_type=_f32)
        hb_ref[...] = _pack(h, F)
        ald_ref[...] = jnp.dot(h, ad_ref[...], preferred_element_type=_f32)

    return pl.pallas_call(
        body,
        grid=(NP // 256,),
        in_specs=[
            pl.BlockSpec((256, K), lambda i: (i, 0)),
            pl.BlockSpec((K, F), lambda i: (0, 0)),
            pl.BlockSpec((F, 16), lambda i: (0, 0)),
        ],
        out_specs=[
            pl.BlockSpec((256, _gw(F)), lambda i: (i, 0)),
            pl.BlockSpec((256, 16), lambda i: (i, 0)),
        ],
        out_shape=[
            jax.ShapeDtypeStruct((NP, _gw(F)), _gdt(F)),
            jax.ShapeDtypeStruct((NP, 16), _f32),
        ],
        interpret=interpret,
    )


def _make_mm_next(K, F, interpret=False):
    def body(u_ref, s_ref, r_ref, b_ref, w_ref, ad_ref, hb_ref, ald_ref):
        sexp = jnp.dot(s_ref[...], r_ref[...],
                       preferred_element_type=_f32) + 1e-16
        xv = jnp.maximum(u_ref[...] / sexp + b_ref[...], 0.0)
        h = jnp.dot(xv.astype(_bf16), w_ref[...].astype(_bf16),
                    preferred_element_type=_f32)
        hb_ref[...] = _pack(h, F)
        ald_ref[...] = jnp.dot(h, ad_ref[...], preferred_element_type=_f32)

    return pl.pallas_call(
        body,
        grid=(NP // 256,),
        in_specs=[
            pl.BlockSpec((256, K), lambda i: (i, 0)),
            pl.BlockSpec((256, 16), lambda i: (i, 0)),
            pl.BlockSpec((16, K), lambda i: (0, 0)),
            pl.BlockSpec((1, K), lambda i: (0, 0)),
            pl.BlockSpec((K, F), lambda i: (0, 0)),
            pl.BlockSpec((F, 16), lambda i: (0, 0)),
        ],
        out_specs=[
            pl.BlockSpec((256, _gw(F)), lambda i: (i, 0)),
            pl.BlockSpec((256, 16), lambda i: (i, 0)),
        ],
        out_shape=[
            jax.ShapeDtypeStruct((NP, _gw(F)), _gdt(F)),
            jax.ShapeDtypeStruct((NP, 16), _f32),
        ],
        interpret=interpret,
    )


def _sc_mesh():
    return plsc.VectorSubcoreMesh(core_axis_name="c", subcore_axis_name="s",
                                  num_cores=2, num_subcores=16)


def _sc_gather_rows(table, srcs, F, G, dtype=_f32):
    nch = RPW // G
    assert nch % 2 == 0
    npairs = nch // 2

    def body(tab_hbm, src_hbm, out_hbm, idx0, idx1, rows0, rows1, sem0, sem1):
        wid = lax.axis_index("s") * 2 + lax.axis_index("c")
        base = wid * RPW

        pltpu.sync_copy(src_hbm.at[pl.ds(base, G)], idx0)
        pltpu.async_copy(tab_hbm.at[idx0], rows0, sem0)

        def pair(cc, carry):
            c = cc * 2
            b0 = base + c * G
            b1 = b0 + G
            pltpu.make_async_copy(tab_hbm.at[idx0], rows0, sem0).wait()
            pltpu.sync_copy(src_hbm.at[pl.ds(b1, G)], idx1)
            pltpu.async_copy(tab_hbm.at[idx1], rows1, sem1)
            pltpu.sync_copy(rows0, out_hbm.at[pl.ds(b0, G)])
            pltpu.make_async_copy(tab_hbm.at[idx1], rows1, sem1).wait()

            @pl.when(cc + 1 < npairs)
            def _():
                b2 = b0 + 2 * G
                pltpu.sync_copy(src_hbm.at[pl.ds(b2, G)], idx0)
                pltpu.async_copy(tab_hbm.at[idx0], rows0, sem0)

            pltpu.sync_copy(rows1, out_hbm.at[pl.ds(b1, G)])
            return carry

        lax.fori_loop(0, npairs, pair, 0)

    f = pl.kernel(
        body,
        out_type=jax.ShapeDtypeStruct((EPa, F), dtype),
        mesh=_sc_mesh(),
        scratch_types=[
            pltpu.VMEM((G,), _i32),
            pltpu.VMEM((G,), _i32),
            pltpu.VMEM((G, F), dtype),
            pltpu.VMEM((G, F), dtype),
            pltpu.SemaphoreType.DMA,
            pltpu.SemaphoreType.DMA,
        ],
    )
    return f(table, srcs)


def _make_seg(F, interpret=False):
    def body(off_ref, dst_hbm, v_hbm, ald_ref, as_ref, r_ref, u_ref, s_ref,
             dstv0, dstv1, vv0, vv1, uacc, sacc, sd0, sv0, sd1, sv1):
        t = pl.program_id(0)
        start = (off_ref[t] // EC) * EC
        end = off_ref[t + 1]
        nch = (end - start + EC - 1) // EC
        rowids = t * TN + lax.broadcasted_iota(_i32, (TN, EC), 0)
        uacc[...] = jnp.zeros((TN, F), _f32)
        sacc[...] = jnp.zeros((TN, 16), _f32)
        rmat = r_ref[...].astype(_bf16)
        aldt = ald_ref[...].astype(_bf16)
        asm = as_ref[...].astype(_bf16)

        def copies(c, dbuf, vbuf, sd, sv):
            base = pl.multiple_of(start + c * EC, EC)
            return (pltpu.make_async_copy(dst_hbm.at[:, pl.ds(base, EC)],
                                          dbuf, sd),
                    pltpu.make_async_copy(v_hbm.at[pl.ds(base, EC), :],
                                          vbuf, sv))

        def load(c, dbuf, vbuf, sd, sv):
            cd, cv = copies(c, dbuf, vbuf, sd, sv)
            cd.start()
            cv.start()

        def waitb(c, dbuf, vbuf, sd, sv):
            cd, cv = copies(c, dbuf, vbuf, sd, sv)
            cd.wait()
            cv.wait()

        def compute(dbuf, vbuf):
            vf = _unpack(vbuf[...], F) if _packed(F) else vbuf[...]
            vb = vf.astype(_bf16)
            maskb = (rowids == dbuf[...]).astype(_bf16)
            als_e = jnp.dot(vb, asm, preferred_element_type=_f32)
            ald_e = lax.dot_general(maskb, aldt, (((0,), (0,)), ((), ())),
                                    preferred_element_type=_f32)
            alpha = als_e + ald_e
            alpha = jnp.where(alpha >= 0, alpha, 0.2 * alpha)
            e = jnp.exp(alpha)
            eb = e.astype(_bf16)
            sacc[...] += jnp.dot(maskb, eb, preferred_element_type=_f32)
            eexp = jnp.dot(eb, rmat, preferred_element_type=_f32)
            vw = (vf * eexp).astype(_bf16)
            uacc[...] += jnp.dot(maskb, vw, preferred_element_type=_f32)

        @pl.when(nch > 0)
        def _():
            load(0, dstv0, vv0, sd0, sv0)

        def pair(cc, carry):
            c0 = 2 * cc
            waitb(c0, dstv0, vv0, sd0, sv0)

            @pl.when(c0 + 1 < nch)
            def _():
                load(c0 + 1, dstv1, vv1, sd1, sv1)

            compute(dstv0, vv0)

            @pl.when(c0 + 1 < nch)
            def _():
                waitb(c0 + 1, dstv1, vv1, sd1, sv1)

                @pl.when(c0 + 2 < nch)
                def _():
                    load(c0 + 2, dstv0, vv0, sd0, sv0)

                compute(dstv1, vv1)

            return carry

        lax.fori_loop(0, (nch + 1) // 2, pair, 0)
        u_ref[...] = uacc[...]
        s_ref[...] = sacc[...]

    grid_spec = pltpu.PrefetchScalarGridSpec(
        num_scalar_prefetch=1,
        grid=(NT,),
        in_specs=[
            pl.BlockSpec(memory_space=pltpu.MemorySpace.HBM),
            pl.BlockSpec(memory_space=pltpu.MemorySpace.HBM),
            pl.BlockSpec((TN, 16), lambda i, *_: (i, 0)),
            pl.BlockSpec((F, 16), lambda i, *_: (0, 0)),
            pl.BlockSpec((16, F), lambda i, *_: (0, 0)),
        ],
        out_specs=[
            pl.BlockSpec((TN, F), lambda i, *_: (i, 0)),
            pl.BlockSpec((TN, 16), lambda i, *_: (i, 0)),
        ],
        scratch_shapes=[
            pltpu.VMEM((1, EC), _i32),
            pltpu.VMEM((1, EC), _i32),
            pltpu.VMEM((EC, _gw(F)), _gdt(F)),
            pltpu.VMEM((EC, _gw(F)), _gdt(F)),
            pltpu.VMEM((TN, F), _f32),
            pltpu.VMEM((TN, 16), _f32),
            pltpu.SemaphoreType.DMA,
            pltpu.SemaphoreType.DMA,
            pltpu.SemaphoreType.DMA,
            pltpu.SemaphoreType.DMA,
        ],
    )
    return pl.pallas_call(
        body,
        grid_spec=grid_spec,
        out_shape=[
            jax.ShapeDtypeStruct((NP, F), _f32),
            jax.ShapeDtypeStruct((NP, 16), _f32),
        ],
        interpret=interpret,
    )


def _make_pool(interpret=False):
    def body(boff_ref, u_ref, s_ref, b_ref, g_ref, xs):
        sden = s_ref[:, 0:1] + 1e-16
        xs[...] = jnp.maximum(u_ref[...] / sden + b_ref[...], 0.0)
        iota8 = lax.broadcasted_iota(_i32, (8, 128), 0)

        def g_body(g, carry):
            st = boff_ref[g]
            en = boff_ref[g + 1]
            nch = (en - st + 7) // 8

            def c_body(c, acc):
                r0 = st + c * 8
                rows = xs[pl.ds(r0, 8), :]
                valid = (r0 + iota8) < en
                return jnp.maximum(acc, jnp.where(valid, rows, -1e30))

            acc = lax.fori_loop(0, nch, c_body,
                                jnp.full((8, 128), -1e30, _f32))
            g_ref[pl.ds(g, 1), :] = jnp.max(acc, axis=0, keepdims=True)
            return carry

        lax.fori_loop(0, B, g_body, 0)

    grid_spec = pltpu.PrefetchScalarGridSpec(
        num_scalar_prefetch=1,
        grid=(1,),
        in_specs=[
            pl.BlockSpec((NP, 128), lambda i, *_: (0, 0)),
            pl.BlockSpec((NP, 16), lambda i, *_: (0, 0)),
            pl.BlockSpec((1, 128), lambda i, *_: (0, 0)),
        ],
        out_specs=[pl.BlockSpec((B, 128), lambda i, *_: (0, 0))],
        scratch_shapes=[pltpu.VMEM((NP, 128), _f32)],
    )
    return pl.pallas_call(
        body,
        grid_spec=grid_spec,
        out_shape=[jax.ShapeDtypeStruct((B, 128), _f32)],
        interpret=interpret,
    )


def _make_mlp(interpret=False):
    def body(g_ref, w_ref, z_ref, se_ref, W5, b5, W6, b6, Ww1, bw1, Ww2, bw2,
             Wz1, bz1, Wz2, bz2, Ws1, bs1, Ws2, bs2, freq_ref, dv_ref,
             sv_ref):
        dot = lambda a, bb: jnp.dot(a, bb, preferred_element_type=_f32)
        relu = lambda v: jnp.maximum(v, 0.0)
        g2 = relu(dot(relu(dot(g_ref[...], W5[...]) + b5[...]), W6[...])
                  + b6[...])
        wv = relu(dot(relu(dot(w_ref[...], Ww1[...]) + bw1[...]), Ww2[...])
                  + bw2[...])
        zv = relu(dot(relu(dot(z_ref[...], Wz1[...]) + bz1[...]), Wz2[...])
                  + bz2[...])
        xwz = jnp.tanh(jnp.maximum(jnp.maximum(g2, wv), zv))
        dv = xwz / jnp.maximum(
            jnp.sqrt(jnp.sum(xwz * xwz, axis=1, keepdims=True)), 1e-12)
        sd = jnp.tanh(dot(relu(dot(se_ref[...], Ws1[...]) + bs1[...]),
                          Ws2[...]) + bs2[...])
        sv = sd / jnp.maximum(
            jnp.sqrt(jnp.sum(sd * sd, axis=1, keepdims=True)), 1e-12)
        freq_ref[...] = 5.0 * lax.dot_general(
            dv, sv, (((1,), (1,)), ((), ())), preferred_element_type=_f32)
        dv_ref[...] = dv
        sv_ref[...] = sv

    full = lambda shape: pl.BlockSpec(shape, lambda: tuple(0 for _ in shape))
    in_shapes = [(B, 128), (B, 256), (B, 512), (SIDE_NP, SIDE_D),
                 (128, 64), (1, 64), (64, 64), (1, 64),
                 (256, 256), (1, 256), (256, 64), (1, 64),
                 (512, Z1P), (1, Z1P), (Z1P, 64), (1, 64),
                 (SIDE_D, 64), (1, 64), (64, 64), (1, 64)]
    return pl.pallas_call(
        body,
        in_specs=[full(s) for s in in_shapes],
        out_specs=[full((B, SIDE_NP)), full((B, 64)), full((SIDE_NP, 64))],
        out_shape=[
            jax.ShapeDtypeStruct((B, SIDE_NP), _f32),
            jax.ShapeDtypeStruct((B, 64), _f32),
            jax.ShapeDtypeStruct((SIDE_NP, 64), _f32),
        ],
        interpret=interpret,
    )


def kernel(x, edge_index, batch, w, z, side_effects, params):
    p = params
    loop = jnp.arange(N, dtype=_i32)
    src = jnp.concatenate([edge_index[0].astype(_i32), loop])
    dst = jnp.concatenate([edge_index[1].astype(_i32), loop])
    order = jnp.argsort(dst)
    srcs = src[order]
    dsts = dst[order]
    pad = jnp.full((EPa - E2,), NP - 1, _i32)
    srcs_p = jnp.concatenate([srcs, pad])
    dsts_p = jnp.concatenate([dsts, pad])
    off = jnp.searchsorted(dsts, jnp.arange(NT + 1) * TN).astype(_i32)
    dst2d = dsts_p.reshape(1, EPa)
    boff = jnp.searchsorted(batch, jnp.arange(B + 1)).astype(_i32)
    xp = jnp.zeros((NP, 128), _f32).at[:N].set(x)

    Wl = (p['W1'], p['W2'], p['W3'], p['W4'])
    Al = ((p['a1s'], p['a1d']), (p['a2s'], p['a2d']), (p['a3s'], p['a3d']),
          (p['a4s'], p['a4d']))
    bl = (p['b1'], p['b2'], p['b3'], p['b4'])

    u = s = None
    for li, (K, F, heads, ocw) in enumerate(LAYERS):
        As, Ad = _att_mats(Al[li][0], Al[li][1], F, heads, ocw)
        R = _rep_mat(F, ocw)
        if li == 0:
            hb, ald = _make_mm_first(K, F)(xp, Wl[0], Ad)
        else:
            Rprev = _rep_mat(K, LAYERS[li - 1][3])
            hb, ald = _make_mm_next(K, F)(
                u, s, Rprev, bl[li - 1].reshape(1, K), Wl[li], Ad)
        G = 112 if F >= 1024 else (168 if F >= 512 else 448)
        V = _sc_gather_rows(hb, srcs_p, _gw(F), G, _gdt(F))
        u, s = _make_seg(F)(off, dst2d, V, ald, As, R)

    g = _make_pool()(boff, u, s, bl[3].reshape(1, 128))[0]

    Wz1p = jnp.zeros((512, Z1P), _f32).at[:, :1600].set(p['Wz1'])
    bz1p = jnp.zeros((1, Z1P), _f32).at[:, :1600].set(p['bz1'][None])
    Wz2p = jnp.zeros((Z1P, 64), _f32).at[:1600].set(p['Wz2'])
    sep = jnp.zeros((SIDE_NP, SIDE_D), _f32).at[:SIDE_N].set(side_effects)

    freq_p, dv, sv_p = _make_mlp()(
        g, w, z, sep,
        p['W5'], p['b5'].reshape(1, 64), p['W6'], p['b6'].reshape(1, 64),
        p['Ww1'], p['bw1'].reshape(1, 256), p['Ww2'], p['bw2'].reshape(1, 64),
        Wz1p, bz1p, Wz2p, p['bz2'].reshape(1, 64),
        p['Ws1'], p['bs1'].reshape(1, 64), p['Ws2'], p['bs2'].reshape(1, 64))
    return freq_p[:, :SIDE_N], dv, sv_p[:SIDE_N]
